# Initial kernel scaffold; baseline (speedup 1.0000x reference)
import jax
import jax.numpy as jnp
from jax.experimental import pallas as pl


def kernel(x, edge_index, edge_w, W, b):
    raise NotImplementedError("write your pallas kernel here")


# speedup vs baseline: 4.1346x; 4.1346x over previous
import functools

import jax
import jax.numpy as jnp
from jax import lax
from jax.experimental import pallas as pl
from jax.experimental.pallas import tpu as pltpu
from jax.experimental.pallas import tpu_sc as plsc

_N = 10000
_E = 320000
_D = 128

_NC = 2
_NS = 16
_L = 16
_NW = _NC * _NS
_EPW = _E // _NW
_CH = 80
_NCHUNK = _EPW // _CH
_RPT = (_N // _NS) & ~7
_RTAIL = _N - _NS * _RPT


def _linear_kernel(x_ref, w_ref, b_ref, out_ref):
    acc = lax.dot_general(x_ref[...], w_ref[...],
                          (((1,), (1,)), ((), ())),
                          preferred_element_type=jnp.float32)
    out_ref[...] = jnp.maximum(acc + b_ref[...][None, :], 0.0)


def _combine_kernel(p_ref, out_ref):
    out_ref[...] = p_ref[0] + p_ref[1]


def _sc_edge_kernel(h_hbm, src_hbm, dst_hbm, w_hbm, z_hbm, out_hbm,
                    src_v, dst_v, w_v, rows_v, acc_sh, sem):
    c = lax.axis_index("c")
    s = lax.axis_index("s")
    wid = s * _NC + c

    pltpu.sync_copy(z_hbm.at[pl.ds(s * _RPT, _RPT)],
                    acc_sh.at[pl.ds(s * _RPT, _RPT)])

    @pl.when(s == _NS - 1)
    def _zero_tail():
        pltpu.sync_copy(z_hbm.at[pl.ds(_NS * _RPT, _RTAIL)],
                        acc_sh.at[pl.ds(_NS * _RPT, _RTAIL)])

    plsc.subcore_barrier()

    base = wid * _EPW

    @pl.loop(0, _NCHUNK)
    def _chunk(i):
        off = base + i * _CH
        pltpu.sync_copy(src_hbm.at[pl.ds(off, _CH)], src_v)
        pltpu.sync_copy(dst_hbm.at[pl.ds(off, _CH)], dst_v)
        pltpu.sync_copy(w_hbm.at[pl.ds(off, _CH)], w_v)
        pltpu.async_copy(h_hbm.at[src_v], rows_v, sem).wait()

        @pl.loop(0, _CH // _L)
        def _group(g):
            w16 = w_v[pl.ds(g * _L, _L)]
            for e in range(_L):
                wv = w16[e]
                for f in range(_D // _L):
                    sl = pl.ds(f * _L, _L)
                    rows_v[g * _L + e, sl] = rows_v[g * _L + e, sl] * wv

        pltpu.sync_copy(rows_v, acc_sh.at[dst_v], add=True)

    plsc.subcore_barrier()
    pltpu.sync_copy(acc_sh.at[pl.ds(s * _RPT, _RPT)],
                    out_hbm.at[c, pl.ds(s * _RPT, _RPT)])

    @pl.when(s == _NS - 1)
    def _out_tail():
        pltpu.sync_copy(acc_sh.at[pl.ds(_NS * _RPT, _RTAIL)],
                        out_hbm.at[c, pl.ds(_NS * _RPT, _RTAIL)])


@functools.lru_cache(maxsize=None)
def _sc_edge():
    return pl.kernel(
        _sc_edge_kernel,
        out_type=jax.ShapeDtypeStruct((_NC, _N, _D), jnp.float32),
        mesh=plsc.VectorSubcoreMesh(core_axis_name="c", subcore_axis_name="s",
                                    num_cores=_NC, num_subcores=_NS),
        scratch_types=[
            pltpu.VMEM((_CH,), jnp.int32),
            pltpu.VMEM((_CH,), jnp.int32),
            pltpu.VMEM((_CH,), jnp.float32),
            pltpu.VMEM((_CH, _D), jnp.float32),
            pltpu.VMEM_SHARED((_N, _D), jnp.float32),
            pltpu.SemaphoreType.DMA,
        ],
    )


def kernel(x, edge_index, edge_w, W, b):
    h = pl.pallas_call(
        _linear_kernel,
        out_shape=jax.ShapeDtypeStruct((_N, _D), jnp.float32),
    )(x, W, b)

    zeros = jnp.zeros((_N, _D), jnp.float32)
    partials = _sc_edge()(h, edge_index[0], edge_index[1], edge_w, zeros)

    out = pl.pallas_call(
        _combine_kernel,
        out_shape=jax.ShapeDtypeStruct((_N, _D), jnp.float32),
    )(partials)
    return out
